# baseline (speedup 1.0000x reference)
import functools

import jax
import jax.numpy as jnp
from jax.experimental import pallas as pl
from jax.experimental.pallas import tpu as pltpu

NSAMPLE = 32
RADIUS2 = 0.8 * 0.8
_BIG = 3.0e38


def _vsa_kernel(kp_ref, xt_ref, table_ref, w1_ref, b1_ref, w2_ref, b2_ref,
                out_ref, d2_ref, g_ref, m_ref):
    QB = kp_ref.shape[0]
    N = xt_ref.shape[1]
    D = table_ref.shape[1]

    kp = kp_ref[...]
    k0 = kp[:, 0:1]
    k1 = kp[:, 1:2]
    k2 = kp[:, 2:3]
    x0 = xt_ref[0:1, :]
    x1 = xt_ref[1:2, :]
    x2 = xt_ref[2:3, :]
    kk = k0 * k0 + k1 * k1 + k2 * k2
    xx = x0 * x0 + x1 * x1 + x2 * x2
    kx = jax.lax.dot(kp, xt_ref[...])
    d2_ref[...] = (kk - 2.0 * kx) + xx

    kpad = jnp.concatenate(
        [kp, jnp.zeros((QB, D - 3), dtype=jnp.float32)], axis=1)

    def body(t, _):
        iota = jax.lax.broadcasted_iota(jnp.int32, (QB, N), 1)
        d2 = d2_ref[...]
        m = jnp.min(d2, axis=1, keepdims=True)
        am = jnp.min(jnp.where(d2 == m, iota, 2147483647),
                     axis=1, keepdims=True)
        onehot = (iota == am)
        d2_ref[...] = jnp.where(onehot, _BIG, d2)
        row = jax.lax.dot(onehot.astype(jnp.float32), table_ref[...],
                          precision=jax.lax.Precision.HIGHEST)
        maskrow = (m <= RADIUS2).astype(jnp.float32)
        base = pl.multiple_of(t * QB, QB)
        g_ref[pl.ds(base, QB), :] = (row - kpad) * maskrow
        m_ref[pl.ds(base, QB), :] = maskrow
        return 0

    jax.lax.fori_loop(0, NSAMPLE, body, 0)

    g = g_ref[...]
    h = jax.lax.dot(g, w1_ref[...],
                    precision=jax.lax.Precision.HIGHEST) + b1_ref[...]
    h = jnp.maximum(h, 0.0)
    h = jax.lax.dot(h, w2_ref[...],
                    precision=jax.lax.Precision.HIGHEST) + b2_ref[...]
    h = jnp.maximum(h, 0.0)
    h = jnp.where(m_ref[...] > 0.0, h, -jnp.inf)
    h3 = h.reshape(NSAMPLE, QB, h.shape[1])
    out = jnp.max(h3, axis=0)
    out_ref[...] = jnp.where(jnp.isfinite(out), out, 0.0)


def kernel(keypoints, xyz, features, W1, b1, W2, b2):
    Q = keypoints.shape[0]
    N = xyz.shape[0]
    C = features.shape[1]
    H = W1.shape[1]
    QB = 64 if Q % 64 == 0 else Q
    D = 48

    xt = xyz.T
    table = jnp.concatenate(
        [xyz, features,
         jnp.zeros((N, D - 3 - C), dtype=features.dtype)], axis=1)
    w1p = jnp.concatenate(
        [W1, jnp.zeros((D - 3 - C, H), dtype=W1.dtype)], axis=0)

    grid = (Q // QB,)
    out = pl.pallas_call(
        _vsa_kernel,
        grid=grid,
        in_specs=[
            pl.BlockSpec((QB, 3), lambda i: (i, 0)),
            pl.BlockSpec((3, N), lambda i: (0, 0)),
            pl.BlockSpec((N, D), lambda i: (0, 0)),
            pl.BlockSpec((D, H), lambda i: (0, 0)),
            pl.BlockSpec((1, H), lambda i: (0, 0)),
            pl.BlockSpec((H, H), lambda i: (0, 0)),
            pl.BlockSpec((1, H), lambda i: (0, 0)),
        ],
        out_specs=pl.BlockSpec((QB, H), lambda i: (i, 0)),
        out_shape=jax.ShapeDtypeStruct((Q, H), jnp.float32),
        scratch_shapes=[
            pltpu.VMEM((QB, N), jnp.float32),
            pltpu.VMEM((NSAMPLE * QB, D), jnp.float32),
            pltpu.VMEM((NSAMPLE * QB, 1), jnp.float32),
        ],
    )(keypoints, xt, table, w1p, b1.reshape(1, H), W2, b2.reshape(1, H))
    return out


# speedup vs baseline: 3.5762x; 3.5762x over previous
import functools

import jax
import jax.numpy as jnp
from jax import lax
from jax.experimental import pallas as pl
from jax.experimental.pallas import tpu as pltpu
from jax.experimental.pallas import tpu_sc as plsc

NSAMPLE = 32
RADIUS2 = 0.8 * 0.8
_BIG = 3.0e38

_NC, _NS = 2, 16
_NW = _NC * _NS


def _sel_kernel(kp_ref, xt_ref, idx_ref, msel_ref, d2_ref):
    QB = kp_ref.shape[0]
    N = xt_ref.shape[1]
    kp = kp_ref[...]
    k0 = kp[:, 0:1]
    k1 = kp[:, 1:2]
    k2 = kp[:, 2:3]
    x0 = xt_ref[0:1, :]
    x1 = xt_ref[1:2, :]
    x2 = xt_ref[2:3, :]
    kk = k0 * k0 + k1 * k1 + k2 * k2
    xx = x0 * x0 + x1 * x1 + x2 * x2
    kx = jax.lax.dot(kp, xt_ref[...])
    d2_ref[...] = (kk - 2.0 * kx) + xx

    iota32 = jax.lax.broadcasted_iota(jnp.int32, (QB, NSAMPLE), 1)

    def body(t, carry):
        idxacc, macc = carry
        iota = jax.lax.broadcasted_iota(jnp.int32, (QB, N), 1)
        d2 = d2_ref[...]
        m = jnp.min(d2, axis=1, keepdims=True)
        am = jnp.min(jnp.where(d2 == m, iota, 2147483647),
                     axis=1, keepdims=True)
        d2_ref[...] = jnp.where(iota == am, _BIG, d2)
        idxacc = jnp.where(iota32 == t, am, idxacc)
        macc = jnp.where(iota32 == t, m, macc)
        return idxacc, macc

    idxacc = jnp.zeros((QB, NSAMPLE), jnp.int32)
    macc = jnp.zeros((QB, NSAMPLE), jnp.float32)
    idxacc, macc = jax.lax.fori_loop(0, NSAMPLE, body, (idxacc, macc))
    idx_ref[...] = idxacc
    msel_ref[...] = macc


def _select(keypoints, xt, QB):
    Q = keypoints.shape[0]
    N = xt.shape[1]
    return pl.pallas_call(
        _sel_kernel,
        grid=(Q // QB,),
        in_specs=[
            pl.BlockSpec((QB, 3), lambda i: (i, 0)),
            pl.BlockSpec((3, N), lambda i: (0, 0)),
        ],
        out_specs=[
            pl.BlockSpec((QB, NSAMPLE), lambda i: (i, 0)),
            pl.BlockSpec((QB, NSAMPLE), lambda i: (i, 0)),
        ],
        out_shape=[
            jax.ShapeDtypeStruct((Q, NSAMPLE), jnp.int32),
            jax.ShapeDtypeStruct((Q, NSAMPLE), jnp.float32),
        ],
        scratch_shapes=[pltpu.VMEM((QB, N), jnp.float32)],
    )(keypoints, xt)


def _gather_kernel(table_hbm, idx_hbm, out_hbm, idx_v, rows_v, sem):
    wid = lax.axis_index("s") * _NC + lax.axis_index("c")
    bpw = idx_v.shape[0]
    half = rows_v.shape[0]
    base = wid * bpw
    pltpu.sync_copy(idx_hbm.at[pl.ds(base, bpw)], idx_v)
    for h in range(bpw // half):
        copies = []
        for j in range(half // 128):
            copies.append(pltpu.async_copy(
                table_hbm.at[idx_v.at[pl.ds(h * half + j * 128, 128)]],
                rows_v.at[pl.ds(j * 128, 128)], sem))
        for c in copies:
            c.wait()
        pltpu.sync_copy(rows_v, out_hbm.at[pl.ds(base + h * half, half)])


def _gather(table, flat_idx):
    B = flat_idx.shape[0]
    D = table.shape[1]
    bpw = B // _NW
    mesh = plsc.VectorSubcoreMesh(core_axis_name="c", subcore_axis_name="s")
    k = functools.partial(
        pl.kernel,
        mesh=mesh,
        out_type=jax.ShapeDtypeStruct((B, D), jnp.float32),
        scratch_types=[
            pltpu.VMEM((bpw,), jnp.int32),
            pltpu.VMEM((bpw // 2, D), jnp.float32),
            pltpu.SemaphoreType.DMA,
        ],
    )(_gather_kernel)
    return k(table, flat_idx)


def _mlp_kernel(rows_ref, kprep_ref, mselv_ref, w1_ref, b1_ref, w2_ref,
                b2_ref, out_ref):
    R = rows_ref.shape[0]
    D = rows_ref.shape[1]
    H = w1_ref.shape[1]
    kpad = jnp.concatenate(
        [kprep_ref[...], jnp.zeros((R, D - 3), jnp.float32)], axis=1)
    maskv = (mselv_ref[...] <= RADIUS2)
    g = (rows_ref[...] - kpad) * maskv.astype(jnp.float32)
    h = jax.lax.dot(g, w1_ref[...],
                    precision=jax.lax.Precision.HIGHEST) + b1_ref[...]
    h = jnp.maximum(h, 0.0)
    h = jax.lax.dot(h, w2_ref[...],
                    precision=jax.lax.Precision.HIGHEST) + b2_ref[...]
    h = jnp.maximum(h, 0.0)
    h = jnp.where(maskv, h, -jnp.inf)
    h3 = h.reshape(R // NSAMPLE, NSAMPLE, H)
    out = jnp.max(h3, axis=1)
    out_ref[...] = jnp.where(jnp.isfinite(out), out, 0.0)


def _mlp(rows, kprep, mselv, w1p, b1, W2, b2, QB):
    Q = kprep.shape[0] // NSAMPLE
    D = rows.shape[1]
    H = w1p.shape[1]
    RB = QB * NSAMPLE
    return pl.pallas_call(
        _mlp_kernel,
        grid=(Q // QB,),
        in_specs=[
            pl.BlockSpec((RB, D), lambda i: (i, 0)),
            pl.BlockSpec((RB, 3), lambda i: (i, 0)),
            pl.BlockSpec((RB, 1), lambda i: (i, 0)),
            pl.BlockSpec((D, H), lambda i: (0, 0)),
            pl.BlockSpec((1, H), lambda i: (0, 0)),
            pl.BlockSpec((H, H), lambda i: (0, 0)),
            pl.BlockSpec((1, H), lambda i: (0, 0)),
        ],
        out_specs=pl.BlockSpec((QB, H), lambda i: (i, 0)),
        out_shape=jax.ShapeDtypeStruct((Q, H), jnp.float32),
    )(rows, kprep, mselv, w1p, b1.reshape(1, H), W2, b2.reshape(1, H))


def kernel(keypoints, xyz, features, W1, b1, W2, b2):
    Q = keypoints.shape[0]
    N = xyz.shape[0]
    C = features.shape[1]
    H = W1.shape[1]
    QB = 64 if Q % 64 == 0 else Q
    D = 128

    xt = xyz.T
    table = jnp.concatenate(
        [xyz, features, jnp.zeros((N, D - 3 - C), features.dtype)], axis=1)
    w1p = jnp.concatenate(
        [W1, jnp.zeros((D - 3 - C, H), W1.dtype)], axis=0)

    idx, msel = _select(keypoints, xt, QB)
    rows = _gather(table, idx.reshape(-1))
    kprep = jnp.repeat(keypoints, NSAMPLE, axis=0)
    mselv = msel.reshape(-1, 1)
    return _mlp(rows, kprep, mselv, w1p, b1, W2, b2, QB)


# speedup vs baseline: 10.8107x; 3.0229x over previous
import functools

import jax
import jax.numpy as jnp
from jax import lax
from jax.experimental import pallas as pl
from jax.experimental.pallas import tpu as pltpu
from jax.experimental.pallas import tpu_sc as plsc

NSAMPLE = 32
RADIUS2 = 0.8 * 0.8
_BIG = 3.0e38

_NC, _NS = 2, 16
_NW = _NC * _NS
_L = 16


def _dist_kernel(kp_ref, xt_ref, d2_ref):
    kp = kp_ref[...]
    k0 = kp[:, 0:1]
    k1 = kp[:, 1:2]
    k2 = kp[:, 2:3]
    x0 = xt_ref[0:1, :]
    x1 = xt_ref[1:2, :]
    x2 = xt_ref[2:3, :]
    kk = k0 * k0 + k1 * k1 + k2 * k2
    xx = x0 * x0 + x1 * x1 + x2 * x2
    kx = jax.lax.dot(kp, xt_ref[...])
    d2 = (kk - 2.0 * kx) + xx
    d2_ref[...] = jnp.where(d2 <= RADIUS2, d2, _BIG)


def _dist(keypoints, xt, QB):
    Q = keypoints.shape[0]
    N = xt.shape[1]
    return pl.pallas_call(
        _dist_kernel,
        grid=(Q // QB,),
        in_specs=[
            pl.BlockSpec((QB, 3), lambda i: (i, 0)),
            pl.BlockSpec((3, N), lambda i: (0, 0)),
        ],
        out_specs=pl.BlockSpec((QB, N), lambda i: (i, 0)),
        out_shape=jax.ShapeDtypeStruct((Q, N), jnp.float32),
    )(keypoints, xt)


def _treemin16(load):
    v = [load(c) for c in range(16)]
    for st in (8, 4, 2, 1):
        v = [jnp.minimum(v[i], v[i + st]) for i in range(st)]
    return v[0]


def _sc_kernel(d2_hbm, table_hbm, rows_hbm, msel_hbm,
               row_v, sv_v, l2_v, idx_v, msel_v, rows_v, tmp_v, tmpi,
               sem, sem2):
    wid = lax.axis_index("s") * _NC + lax.axis_index("c")
    iota = lax.iota(jnp.int32, _L)
    perms = [iota ^ sh for sh in (8, 4, 2, 1)]

    def bfmin_f(v):
        for sh in (8, 4, 2, 1):
            tmp_v[pl.ds(0, _L)] = v
            tmp_v[pl.ds(_L, _L)] = v
            v = jnp.minimum(v, tmp_v[pl.ds(sh, _L)])
        return v

    def bfmin_i(v):
        for sh in (8, 4, 2, 1):
            tmpi[pl.ds(0, _L)] = v
            tmpi[pl.ds(_L, _L)] = v
            v = jnp.minimum(v, tmpi[pl.ds(sh, _L)])
        return v

    def per_query(qlocal, _):
        q = wid * 32 + qlocal
        pltpu.sync_copy(d2_hbm.at[q], row_v)

        def build_sv(s, _):
            base = s * 256
            sv_v[pl.ds(s * 16, _L)] = _treemin16(
                lambda c, b=base: row_v[pl.ds(b + c * 16, _L)])
            return 0
        lax.fori_loop(0, 128, build_sv, 0)

        def build_l2(k, _):
            base = k * 256
            l2_v[pl.ds(k * 16, _L)] = _treemin16(
                lambda s2, b=base: sv_v[pl.ds(b + s2 * 16, _L)])
            return 0
        lax.fori_loop(0, 8, build_l2, 0)

        ib0 = qlocal * 32

        def extract(t, _):
            l2 = [l2_v[pl.ds(k * 16, _L)] for k in range(8)]
            g = l2
            for st in (4, 2, 1):
                g = [jnp.minimum(g[i], g[i + st]) for i in range(st)]
            m = bfmin_f(g[0])[0]
            enc1 = [jnp.where(l2[k] == m, iota + k * 16, 9999)
                    for k in range(8)]
            for st in (4, 2, 1):
                enc1 = [jnp.minimum(enc1[i], enc1[i + st])
                        for i in range(st)]
            ks = bfmin_i(enc1[0])[0] // 16

            enc2 = _treemin16(
                lambda s2: jnp.where(
                    sv_v[pl.ds(ks * 256 + s2 * 16, _L)] == m,
                    iota + s2 * 16, 9999))
            sstar = ks * 16 + bfmin_i(enc2)[0] // 16

            enc3 = _treemin16(
                lambda c2: jnp.where(
                    row_v[pl.ds(sstar * 256 + c2 * 16, _L)] == m,
                    iota + c2 * 16, 9999))
            e3 = bfmin_i(enc3)[0]
            lane = e3 % 16
            nstar = sstar * 256 + e3

            cb = sstar * 256 + e3 - lane
            row_v[pl.ds(cb, _L)] = jnp.where(
                iota == lane, _BIG, row_v[pl.ds(cb, _L)])
            sv_v[pl.ds(sstar * 16, _L)] = _treemin16(
                lambda c: row_v[pl.ds(sstar * 256 + c * 16, _L)])
            l2_v[pl.ds(ks * 16, _L)] = _treemin16(
                lambda s2: sv_v[pl.ds(ks * 256 + s2 * 16, _L)])

            idx_v[pl.ds(ib0, _L)] = jnp.where(
                iota == t, nstar, idx_v[pl.ds(ib0, _L)])
            idx_v[pl.ds(ib0 + 16, _L)] = jnp.where(
                iota == t - 16, nstar, idx_v[pl.ds(ib0 + 16, _L)])
            msel_v[pl.ds(ib0, _L)] = jnp.where(
                iota == t, m, msel_v[pl.ds(ib0, _L)])
            msel_v[pl.ds(ib0 + 16, _L)] = jnp.where(
                iota == t - 16, m, msel_v[pl.ds(ib0 + 16, _L)])
            return 0

        lax.fori_loop(0, NSAMPLE, extract, 0)
        return 0

    lax.fori_loop(0, 32, per_query, 0)

    base = wid * 1024
    half = rows_v.shape[0]
    for h in range(1024 // half):
        copies = []
        for j in range(half // 128):
            copies.append(pltpu.async_copy(
                table_hbm.at[idx_v.at[pl.ds(h * half + j * 128, 128)]],
                rows_v.at[pl.ds(j * 128, 128)], sem))
        for c in copies:
            c.wait()
        pltpu.sync_copy(rows_v, rows_hbm.at[pl.ds(base + h * half, half)])
    pltpu.sync_copy(msel_v, msel_hbm.at[pl.ds(base, 1024)])


def _sc_select_gather(d2m, table):
    N, D = table.shape
    B = 1024 * NSAMPLE
    mesh = plsc.VectorSubcoreMesh(core_axis_name="c", subcore_axis_name="s")
    k = functools.partial(
        pl.kernel,
        mesh=mesh,
        out_type=[
            jax.ShapeDtypeStruct((B, D), jnp.float32),
            jax.ShapeDtypeStruct((B,), jnp.float32),
        ],
        scratch_types=[
            pltpu.VMEM((32768,), jnp.float32),
            pltpu.VMEM((2048,), jnp.float32),
            pltpu.VMEM((128,), jnp.float32),
            pltpu.VMEM((1024,), jnp.int32),
            pltpu.VMEM((1024,), jnp.float32),
            pltpu.VMEM((512, 128), jnp.float32),
            pltpu.VMEM((2 * _L,), jnp.float32),
            pltpu.VMEM((2 * _L,), jnp.int32),
            pltpu.SemaphoreType.DMA,
            pltpu.SemaphoreType.DMA,
        ],
    )(_sc_kernel)
    return k(d2m, table)


def _mlp_kernel(rows_ref, kprep_ref, mselv_ref, w1_ref, b1_ref, w2_ref,
                b2_ref, out_ref):
    R = rows_ref.shape[0]
    D = rows_ref.shape[1]
    H = w1_ref.shape[1]
    kpad = jnp.concatenate(
        [kprep_ref[...], jnp.zeros((R, D - 3), jnp.float32)], axis=1)
    maskv = (mselv_ref[...] <= RADIUS2)
    g = (rows_ref[...] - kpad) * maskv.astype(jnp.float32)
    h = jax.lax.dot(g, w1_ref[...],
                    precision=jax.lax.Precision.HIGHEST) + b1_ref[...]
    h = jnp.maximum(h, 0.0)
    h = jax.lax.dot(h, w2_ref[...],
                    precision=jax.lax.Precision.HIGHEST) + b2_ref[...]
    h = jnp.maximum(h, 0.0)
    h = jnp.where(maskv, h, -jnp.inf)
    h3 = h.reshape(R // NSAMPLE, NSAMPLE, H)
    out = jnp.max(h3, axis=1)
    out_ref[...] = jnp.where(jnp.isfinite(out), out, 0.0)


def _mlp(rows, kprep, mselv, w1p, b1, W2, b2, QB):
    Q = kprep.shape[0] // NSAMPLE
    D = rows.shape[1]
    H = w1p.shape[1]
    RB = QB * NSAMPLE
    return pl.pallas_call(
        _mlp_kernel,
        grid=(Q // QB,),
        in_specs=[
            pl.BlockSpec((RB, D), lambda i: (i, 0)),
            pl.BlockSpec((RB, 3), lambda i: (i, 0)),
            pl.BlockSpec((RB, 1), lambda i: (i, 0)),
            pl.BlockSpec((D, H), lambda i: (0, 0)),
            pl.BlockSpec((1, H), lambda i: (0, 0)),
            pl.BlockSpec((H, H), lambda i: (0, 0)),
            pl.BlockSpec((1, H), lambda i: (0, 0)),
        ],
        out_specs=pl.BlockSpec((QB, H), lambda i: (i, 0)),
        out_shape=jax.ShapeDtypeStruct((Q, H), jnp.float32),
    )(rows, kprep, mselv, w1p, b1.reshape(1, H), W2, b2.reshape(1, H))


def kernel(keypoints, xyz, features, W1, b1, W2, b2):
    Q = keypoints.shape[0]
    N = xyz.shape[0]
    C = features.shape[1]
    H = W1.shape[1]
    QB = 64 if Q % 64 == 0 else Q
    D = 128

    xt = xyz.T
    table = jnp.concatenate(
        [xyz, features, jnp.zeros((N, D - 3 - C), features.dtype)], axis=1)
    w1p = jnp.concatenate(
        [W1, jnp.zeros((D - 3 - C, H), W1.dtype)], axis=0)

    d2m = _dist(keypoints, xt, QB)
    rows, mself = _sc_select_gather(d2m, table)
    kprep = jnp.repeat(keypoints, NSAMPLE, axis=0)
    return _mlp(rows, kprep, mself.reshape(-1, 1), w1p, b1, W2, b2, QB)


# speedup vs baseline: 12.2386x; 1.1321x over previous
import functools

import jax
import jax.numpy as jnp
from jax import lax
from jax.experimental import pallas as pl
from jax.experimental.pallas import tpu as pltpu
from jax.experimental.pallas import tpu_sc as plsc

NSAMPLE = 32
RADIUS2 = 0.8 * 0.8
_BIG = 3.0e38

_NC, _NS = 2, 16
_NW = _NC * _NS
_L = 16


def _dist_kernel(kp_ref, xt_ref, d2_ref):
    kp = kp_ref[...]
    k0 = kp[:, 0:1]
    k1 = kp[:, 1:2]
    k2 = kp[:, 2:3]
    x0 = xt_ref[0:1, :]
    x1 = xt_ref[1:2, :]
    x2 = xt_ref[2:3, :]
    kk = k0 * k0 + k1 * k1 + k2 * k2
    xx = x0 * x0 + x1 * x1 + x2 * x2
    kx = jax.lax.dot(kp, xt_ref[...])
    d2 = (kk - 2.0 * kx) + xx
    d2_ref[...] = jnp.where(d2 <= RADIUS2, d2, _BIG)


def _dist(keypoints, xt, QB):
    Q = keypoints.shape[0]
    N = xt.shape[1]
    return pl.pallas_call(
        _dist_kernel,
        grid=(Q // QB,),
        in_specs=[
            pl.BlockSpec((QB, 3), lambda i: (i, 0)),
            pl.BlockSpec((3, N), lambda i: (0, 0)),
        ],
        out_specs=pl.BlockSpec((QB, N), lambda i: (i, 0)),
        out_shape=jax.ShapeDtypeStruct((Q, N), jnp.float32),
    )(keypoints, xt)


def _treemin16(load):
    v = [load(c) for c in range(16)]
    for st in (8, 4, 2, 1):
        v = [jnp.minimum(v[i], v[i + st]) for i in range(st)]
    return v[0]


def _sc_kernel(d2_hbm, table_hbm, rows_hbm, msel_hbm,
               row_v, sv_v, l2_v, idx_v, msel_v, rows_v, tmp_v, tmpi,
               sem, sem2):
    wid = lax.axis_index("s") * _NC + lax.axis_index("c")
    iota = lax.iota(jnp.int32, _L)
    perms = [iota ^ sh for sh in (8, 4, 2, 1)]

    def bfmin_f(v):
        for sh in (8, 4, 2, 1):
            tmp_v[pl.ds(0, _L)] = v
            tmp_v[pl.ds(_L, _L)] = v
            v = jnp.minimum(v, tmp_v[pl.ds(sh, _L)])
        return v

    def bfmin_i(v):
        for sh in (8, 4, 2, 1):
            tmpi[pl.ds(0, _L)] = v
            tmpi[pl.ds(_L, _L)] = v
            v = jnp.minimum(v, tmpi[pl.ds(sh, _L)])
        return v

    pltpu.async_copy(d2_hbm.at[wid * 32], row_v.at[pl.ds(0, 32768)], sem2)

    def per_query(qlocal, _):
        bb = (qlocal % 2) * 32768
        pltpu.make_async_copy(
            d2_hbm.at[0], row_v.at[pl.ds(bb, 32768)], sem2).wait()
        qn = wid * 32 + jnp.minimum(qlocal + 1, 31)
        bbn = ((qlocal + 1) % 2) * 32768
        pltpu.async_copy(d2_hbm.at[qn], row_v.at[pl.ds(bbn, 32768)], sem2)

        def build_sv(s, _):
            base = s * 256
            sv_v[pl.ds(s * 16, _L)] = _treemin16(
                lambda c, b=base: row_v[pl.ds(bb + b + c * 16, _L)])
            return 0
        lax.fori_loop(0, 128, build_sv, 0)

        def build_l2(k, _):
            base = k * 256
            l2_v[pl.ds(k * 16, _L)] = _treemin16(
                lambda s2, b=base: sv_v[pl.ds(b + s2 * 16, _L)])
            return 0
        lax.fori_loop(0, 8, build_l2, 0)

        ib0 = qlocal * 32

        def extract(t, _):
            l2 = [l2_v[pl.ds(k * 16, _L)] for k in range(8)]
            g = l2
            for st in (4, 2, 1):
                g = [jnp.minimum(g[i], g[i + st]) for i in range(st)]
            m = bfmin_f(g[0])[0]
            enc1 = [jnp.where(l2[k] == m, iota + k * 16, 9999)
                    for k in range(8)]
            for st in (4, 2, 1):
                enc1 = [jnp.minimum(enc1[i], enc1[i + st])
                        for i in range(st)]
            ks = bfmin_i(enc1[0])[0] // 16

            enc2 = _treemin16(
                lambda s2: jnp.where(
                    sv_v[pl.ds(ks * 256 + s2 * 16, _L)] == m,
                    iota + s2 * 16, 9999))
            sstar = ks * 16 + bfmin_i(enc2)[0] // 16

            enc3 = _treemin16(
                lambda c2: jnp.where(
                    row_v[pl.ds(bb + sstar * 256 + c2 * 16, _L)] == m,
                    iota + c2 * 16, 9999))
            e3 = bfmin_i(enc3)[0]
            lane = e3 % 16
            nstar = sstar * 256 + e3

            cb = bb + sstar * 256 + e3 - lane
            row_v[pl.ds(cb, _L)] = jnp.where(
                iota == lane, _BIG, row_v[pl.ds(cb, _L)])
            sv_v[pl.ds(sstar * 16, _L)] = _treemin16(
                lambda c: row_v[pl.ds(bb + sstar * 256 + c * 16, _L)])
            l2_v[pl.ds(ks * 16, _L)] = _treemin16(
                lambda s2: sv_v[pl.ds(ks * 256 + s2 * 16, _L)])

            idx_v[pl.ds(ib0, _L)] = jnp.where(
                iota == t, nstar, idx_v[pl.ds(ib0, _L)])
            idx_v[pl.ds(ib0 + 16, _L)] = jnp.where(
                iota == t - 16, nstar, idx_v[pl.ds(ib0 + 16, _L)])
            msel_v[pl.ds(ib0, _L)] = jnp.where(
                iota == t, m, msel_v[pl.ds(ib0, _L)])
            msel_v[pl.ds(ib0 + 16, _L)] = jnp.where(
                iota == t - 16, m, msel_v[pl.ds(ib0 + 16, _L)])
            return 0

        lax.fori_loop(0, NSAMPLE, extract, 0)
        return 0

    lax.fori_loop(0, 32, per_query, 0)
    pltpu.make_async_copy(
        d2_hbm.at[0], row_v.at[pl.ds(0, 32768)], sem2).wait()

    base = wid * 1024
    half = rows_v.shape[0]
    for h in range(1024 // half):
        copies = []
        for j in range(half // 128):
            copies.append(pltpu.async_copy(
                table_hbm.at[idx_v.at[pl.ds(h * half + j * 128, 128)]],
                rows_v.at[pl.ds(j * 128, 128)], sem))
        for c in copies:
            c.wait()
        pltpu.sync_copy(rows_v, rows_hbm.at[pl.ds(base + h * half, half)])
    pltpu.sync_copy(msel_v, msel_hbm.at[pl.ds(base, 1024)])


def _sc_select_gather(d2m, table):
    N, D = table.shape
    B = 1024 * NSAMPLE
    mesh = plsc.VectorSubcoreMesh(core_axis_name="c", subcore_axis_name="s")
    k = functools.partial(
        pl.kernel,
        mesh=mesh,
        out_type=[
            jax.ShapeDtypeStruct((B, D), jnp.float32),
            jax.ShapeDtypeStruct((B,), jnp.float32),
        ],
        scratch_types=[
            pltpu.VMEM((2 * 32768,), jnp.float32),
            pltpu.VMEM((2048,), jnp.float32),
            pltpu.VMEM((128,), jnp.float32),
            pltpu.VMEM((1024,), jnp.int32),
            pltpu.VMEM((1024,), jnp.float32),
            pltpu.VMEM((256, 128), jnp.float32),
            pltpu.VMEM((2 * _L,), jnp.float32),
            pltpu.VMEM((2 * _L,), jnp.int32),
            pltpu.SemaphoreType.DMA,
            pltpu.SemaphoreType.DMA,
        ],
    )(_sc_kernel)
    return k(d2m, table)


def _mlp_kernel(rows_ref, kprep_ref, mselv_ref, w1_ref, b1_ref, w2_ref,
                b2_ref, out_ref):
    R = rows_ref.shape[0]
    D = rows_ref.shape[1]
    H = w1_ref.shape[1]
    kpad = jnp.concatenate(
        [kprep_ref[...], jnp.zeros((R, D - 3), jnp.float32)], axis=1)
    maskv = (mselv_ref[...] <= RADIUS2)
    g = (rows_ref[...] - kpad) * maskv.astype(jnp.float32)
    h = jax.lax.dot(g, w1_ref[...],
                    precision=jax.lax.Precision.HIGHEST) + b1_ref[...]
    h = jnp.maximum(h, 0.0)
    h = jax.lax.dot(h, w2_ref[...],
                    precision=jax.lax.Precision.HIGHEST) + b2_ref[...]
    h = jnp.maximum(h, 0.0)
    h = jnp.where(maskv, h, -jnp.inf)
    h3 = h.reshape(R // NSAMPLE, NSAMPLE, H)
    out = jnp.max(h3, axis=1)
    out_ref[...] = jnp.where(jnp.isfinite(out), out, 0.0)


def _mlp(rows, kprep, mselv, w1p, b1, W2, b2, QB):
    Q = kprep.shape[0] // NSAMPLE
    D = rows.shape[1]
    H = w1p.shape[1]
    RB = QB * NSAMPLE
    return pl.pallas_call(
        _mlp_kernel,
        grid=(Q // QB,),
        in_specs=[
            pl.BlockSpec((RB, D), lambda i: (i, 0)),
            pl.BlockSpec((RB, 3), lambda i: (i, 0)),
            pl.BlockSpec((RB, 1), lambda i: (i, 0)),
            pl.BlockSpec((D, H), lambda i: (0, 0)),
            pl.BlockSpec((1, H), lambda i: (0, 0)),
            pl.BlockSpec((H, H), lambda i: (0, 0)),
            pl.BlockSpec((1, H), lambda i: (0, 0)),
        ],
        out_specs=pl.BlockSpec((QB, H), lambda i: (i, 0)),
        out_shape=jax.ShapeDtypeStruct((Q, H), jnp.float32),
    )(rows, kprep, mselv, w1p, b1.reshape(1, H), W2, b2.reshape(1, H))


def kernel(keypoints, xyz, features, W1, b1, W2, b2):
    Q = keypoints.shape[0]
    N = xyz.shape[0]
    C = features.shape[1]
    H = W1.shape[1]
    QB = 64 if Q % 64 == 0 else Q
    D = 128

    xt = xyz.T
    table = jnp.concatenate(
        [xyz, features, jnp.zeros((N, D - 3 - C), features.dtype)], axis=1)
    w1p = jnp.concatenate(
        [W1, jnp.zeros((D - 3 - C, H), W1.dtype)], axis=0)

    d2m = _dist(keypoints, xt, QB)
    rows, mself = _sc_select_gather(d2m, table)
    kprep = jnp.repeat(keypoints, NSAMPLE, axis=0)
    return _mlp(rows, kprep, mself.reshape(-1, 1), w1p, b1, W2, b2, QB)
